# Initial kernel scaffold; baseline (speedup 1.0000x reference)
import functools

import jax
import jax.numpy as jnp
from jax import lax
from jax.experimental import pallas as pl
from jax.experimental.pallas import tpu as pltpu
from jax.experimental.pallas import tpu_sc as plsc

N = 10000
E = 320000
H = 128
NC = 2
NS = 16
NW = NC * NS
EPW = E // NW
CHUNK = 80
NCHUNK = EPW // CHUNK
RPT = N // NS
ZROWS = 125

_DN = (((1,), (0,)), ((), ()))


def _edge_mlp_body(x_ref, w1_ref, b1_ref, w2_ref, b2_ref, o_ref):
    t = lax.dot_general(x_ref[...], w1_ref[...], _DN,
                        preferred_element_type=jnp.float32,
                        precision=lax.Precision.HIGHEST)
    t = t + b1_ref[...]
    t = t * jax.nn.sigmoid(t)
    u = lax.dot_general(t, w2_ref[...], _DN,
                        preferred_element_type=jnp.float32,
                        precision=lax.Precision.HIGHEST)
    o_ref[...] = u + b2_ref[...]


def _edge_mlp(edge_in, W1e, b1e, W2e, b2e):
    BE = 4000
    grid = (E // BE,)
    return pl.pallas_call(
        _edge_mlp_body,
        grid=grid,
        in_specs=[
            pl.BlockSpec((BE, edge_in.shape[1]), lambda i: (i, 0)),
            pl.BlockSpec(W1e.shape, lambda i: (0, 0)),
            pl.BlockSpec((1, H), lambda i: (0, 0)),
            pl.BlockSpec(W2e.shape, lambda i: (0, 0)),
            pl.BlockSpec((1, H), lambda i: (0, 0)),
        ],
        out_specs=pl.BlockSpec((BE, H), lambda i: (i, 0)),
        out_shape=jax.ShapeDtypeStruct((E, H), jnp.float32),
    )(edge_in, W1e, b1e.reshape(1, H), W2e, b2e.reshape(1, H))


def _sc_messages(h, src_blk, dst_blk, w_edge):
    mesh = plsc.VectorSubcoreMesh(core_axis_name="core", subcore_axis_name="subcore")

    @functools.partial(
        pl.kernel,
        out_type=jax.ShapeDtypeStruct((NC, N, H), jnp.float32),
        mesh=mesh,
        scratch_types=[
            pltpu.VMEM((NCHUNK, CHUNK), jnp.int32),
            pltpu.VMEM((NCHUNK, CHUNK), jnp.int32),
            pltpu.VMEM((CHUNK, H), jnp.float32),
            pltpu.VMEM((CHUNK, H), jnp.float32),
            pltpu.VMEM((ZROWS, H), jnp.float32),
            pltpu.VMEM_SHARED((N, H), jnp.float32),
            pltpu.SemaphoreType.DMA,
            pltpu.SemaphoreType.DMA,
        ],
    )
    def k(h_hbm, src_hbm, dst_hbm, we_hbm, out_hbm,
          src_v, dst_v, rows_v, wv, zero_v, agg, sem1, sem2):
        cid = lax.axis_index("core")
        sid = lax.axis_index("subcore")
        wid = cid * NS + sid

        pltpu.sync_copy(src_hbm.at[wid], src_v)
        pltpu.sync_copy(dst_hbm.at[wid], dst_v)

        @pl.loop(0, ZROWS)
        def _(i):
            for kk in range(H // 16):
                zero_v[i, pl.ds(kk * 16, 16)] = jnp.zeros((16,), jnp.float32)

        for r in range(RPT // ZROWS):
            pltpu.sync_copy(zero_v, agg.at[pl.ds(sid * RPT + r * ZROWS, ZROWS)])
        plsc.subcore_barrier()

        ebase = wid * EPW

        @pl.loop(0, NCHUNK)
        def _(j):
            cp1 = pltpu.async_copy(h_hbm.at[src_v.at[j]], rows_v, sem1)
            cp2 = pltpu.async_copy(we_hbm.at[pl.ds(ebase + j * CHUNK, CHUNK)], wv, sem2)
            cp1.wait()
            cp2.wait()

            @pl.loop(0, CHUNK)
            def _(e):
                for kk in range(H // 16):
                    s = pl.ds(kk * 16, 16)
                    rows_v[e, s] = rows_v[e, s] * wv[e, s]

            pltpu.sync_copy(rows_v, agg.at[dst_v.at[j]], add=True)

        plsc.subcore_barrier()
        for r in range(RPT // ZROWS):
            sl = pl.ds(sid * RPT + r * ZROWS, ZROWS)
            pltpu.sync_copy(agg.at[sl], out_hbm.at[cid, sl])

    return k(h, src_blk, dst_blk, w_edge)


def _node_body(h_ref, p_ref, w1h_ref, w1a_ref, b1_ref, w2_ref, b2_ref,
               g_ref, bb_ref, o_ref):
    hb = h_ref[...]
    agg = p_ref[0] + p_ref[1]
    t = lax.dot_general(hb, w1h_ref[...], _DN,
                        preferred_element_type=jnp.float32,
                        precision=lax.Precision.HIGHEST)
    t = t + lax.dot_general(agg, w1a_ref[...], _DN,
                            preferred_element_type=jnp.float32,
                            precision=lax.Precision.HIGHEST)
    t = t + b1_ref[...]
    t = t * jax.nn.sigmoid(t)
    u = lax.dot_general(t, w2_ref[...], _DN,
                        preferred_element_type=jnp.float32,
                        precision=lax.Precision.HIGHEST)
    x = hb + u + b2_ref[...]
    mean = jnp.mean(x, axis=1, keepdims=True)
    var = jnp.mean((x - mean) ** 2, axis=1, keepdims=True)
    o_ref[...] = (x - mean) / jnp.sqrt(var + 1e-5) * g_ref[...] + bb_ref[...]


def _node_update(h, partials, W1n, b1n, W2n, b2n, gamma, beta):
    BN = 2000
    grid = (N // BN,)
    w1h = W1n[:H]
    w1a = W1n[H:]
    return pl.pallas_call(
        _node_body,
        grid=grid,
        in_specs=[
            pl.BlockSpec((BN, H), lambda i: (i, 0)),
            pl.BlockSpec((NC, BN, H), lambda i: (0, i, 0)),
            pl.BlockSpec((H, H), lambda i: (0, 0)),
            pl.BlockSpec((H, H), lambda i: (0, 0)),
            pl.BlockSpec((1, H), lambda i: (0, 0)),
            pl.BlockSpec((H, H), lambda i: (0, 0)),
            pl.BlockSpec((1, H), lambda i: (0, 0)),
            pl.BlockSpec((1, H), lambda i: (0, 0)),
            pl.BlockSpec((1, H), lambda i: (0, 0)),
        ],
        out_specs=pl.BlockSpec((BN, H), lambda i: (i, 0)),
        out_shape=jax.ShapeDtypeStruct((N, H), jnp.float32),
    )(h, partials, w1h, w1a, b1n.reshape(1, H), W2n, b2n.reshape(1, H),
      gamma.reshape(1, H), beta.reshape(1, H))


def kernel(h, edge_index, edge_attr, edge_sh,
           W1e, b1e, W2e, b2e, W1n, b1n, W2n, b2n, gamma, beta):
    edge_in = jnp.concatenate([edge_attr, edge_sh], axis=1)
    src_blk = edge_index[0].reshape(NW, NCHUNK, CHUNK)
    dst_blk = edge_index[1].reshape(NW, NCHUNK, CHUNK)

    w_edge = _edge_mlp(edge_in, W1e, b1e, W2e, b2e)
    partials = _sc_messages(h, src_blk, dst_blk, w_edge)
    return _node_update(h, partials, W1n, b1n, W2n, b2n, gamma, beta)


# speedup vs baseline: 1.8832x; 1.8832x over previous
import functools

import jax
import jax.numpy as jnp
from jax import lax
from jax.experimental import pallas as pl
from jax.experimental.pallas import tpu as pltpu
from jax.experimental.pallas import tpu_sc as plsc

N = 10000
E = 320000
H = 128
NC = 2
NS = 16
NW = NC * NS
EPW = E // NW
CHUNK = 80
NCHUNK = EPW // CHUNK
IBLK = 25
NIB = NCHUNK // IBLK
RPT = 624
REM = N - RPT * NS

_DN = (((1,), (0,)), ((), ()))


def _edge_mlp_body(x_ref, w1_ref, b1_ref, w2_ref, b2_ref, o_ref):
    t = lax.dot_general(x_ref[...], w1_ref[...], _DN,
                        preferred_element_type=jnp.float32,
                        precision=lax.Precision.HIGHEST)
    t = t + b1_ref[...]
    t = t * jax.nn.sigmoid(t)
    u = lax.dot_general(t, w2_ref[...], _DN,
                        preferred_element_type=jnp.float32,
                        precision=lax.Precision.HIGHEST)
    o_ref[...] = u + b2_ref[...]


def _edge_mlp(edge_in, W1e, b1e, W2e, b2e):
    BE = 4000
    grid = (E // BE,)
    return pl.pallas_call(
        _edge_mlp_body,
        grid=grid,
        in_specs=[
            pl.BlockSpec((BE, edge_in.shape[1]), lambda i: (i, 0)),
            pl.BlockSpec(W1e.shape, lambda i: (0, 0)),
            pl.BlockSpec((1, H), lambda i: (0, 0)),
            pl.BlockSpec(W2e.shape, lambda i: (0, 0)),
            pl.BlockSpec((1, H), lambda i: (0, 0)),
        ],
        out_specs=pl.BlockSpec((BE, H), lambda i: (i, 0)),
        out_shape=jax.ShapeDtypeStruct((E, H), jnp.float32),
    )(edge_in, W1e, b1e.reshape(1, H), W2e, b2e.reshape(1, H))


def _sc_messages(h, src_blk, dst_blk, w_edge):
    mesh = plsc.VectorSubcoreMesh(core_axis_name="core", subcore_axis_name="subcore")

    @functools.partial(
        pl.kernel,
        out_type=jax.ShapeDtypeStruct((NC, N, H), jnp.float32),
        mesh=mesh,
        scratch_types=[
            pltpu.VMEM((IBLK, CHUNK), jnp.int32),
            pltpu.VMEM((IBLK, CHUNK), jnp.int32),
            pltpu.VMEM((CHUNK, H), jnp.float32),
            pltpu.VMEM((CHUNK, H), jnp.float32),
            pltpu.VMEM_SHARED((N, H), jnp.float32),
            pltpu.SemaphoreType.DMA,
            pltpu.SemaphoreType.DMA,
        ],
    )
    def k(h_hbm, src_hbm, dst_hbm, we_hbm, out_hbm,
          src_v, dst_v, rows_v, wv, agg, sem1, sem2):
        cid = lax.axis_index("core")
        sid = lax.axis_index("subcore")
        wid = cid * NS + sid

        @pl.loop(0, CHUNK)
        def _(i):
            for kk in range(H // 16):
                rows_v[i, pl.ds(kk * 16, 16)] = jnp.zeros((16,), jnp.float32)

        zbase = pl.multiple_of(sid * RPT, 8)
        for r in range(7):
            pltpu.sync_copy(rows_v, agg.at[pl.ds(zbase + r * CHUNK, CHUNK)])
        pltpu.sync_copy(rows_v.at[pl.ds(0, RPT - 7 * CHUNK)],
                        agg.at[pl.ds(zbase + 7 * CHUNK, RPT - 7 * CHUNK)])

        @pl.when(sid == NS - 1)
        def _():
            pltpu.sync_copy(rows_v.at[pl.ds(0, REM)],
                            agg.at[pl.ds(NS * RPT, REM)])

        plsc.subcore_barrier()

        for b in range(NIB):
            pltpu.sync_copy(src_hbm.at[wid, b], src_v)
            pltpu.sync_copy(dst_hbm.at[wid, b], dst_v)

            @pl.loop(0, IBLK)
            def _(jj):
                cp1 = pltpu.async_copy(h_hbm.at[src_v.at[jj]], rows_v, sem1)
                cp2 = pltpu.async_copy(
                    we_hbm.at[pl.ds(wid * EPW + b * IBLK * CHUNK + jj * CHUNK,
                                    CHUNK)],
                    wv, sem2)
                cp1.wait()
                cp2.wait()

                @pl.loop(0, CHUNK)
                def _(e):
                    for kk in range(H // 16):
                        s = pl.ds(kk * 16, 16)
                        rows_v[e, s] = rows_v[e, s] * wv[e, s]

                pltpu.sync_copy(rows_v, agg.at[dst_v.at[jj]], add=True)

        plsc.subcore_barrier()
        sl = pl.ds(zbase, RPT)
        pltpu.sync_copy(agg.at[sl], out_hbm.at[cid, sl])

        @pl.when(sid == NS - 1)
        def _():
            slr = pl.ds(NS * RPT, REM)
            pltpu.sync_copy(agg.at[slr], out_hbm.at[cid, slr])

    return k(h, src_blk, dst_blk, w_edge)


def _node_body(h_ref, p_ref, w1h_ref, w1a_ref, b1_ref, w2_ref, b2_ref,
               g_ref, bb_ref, o_ref):
    hb = h_ref[...]
    agg = p_ref[0] + p_ref[1]
    t = lax.dot_general(hb, w1h_ref[...], _DN,
                        preferred_element_type=jnp.float32,
                        precision=lax.Precision.HIGHEST)
    t = t + lax.dot_general(agg, w1a_ref[...], _DN,
                            preferred_element_type=jnp.float32,
                            precision=lax.Precision.HIGHEST)
    t = t + b1_ref[...]
    t = t * jax.nn.sigmoid(t)
    u = lax.dot_general(t, w2_ref[...], _DN,
                        preferred_element_type=jnp.float32,
                        precision=lax.Precision.HIGHEST)
    x = hb + u + b2_ref[...]
    mean = jnp.mean(x, axis=1, keepdims=True)
    var = jnp.mean((x - mean) ** 2, axis=1, keepdims=True)
    o_ref[...] = (x - mean) / jnp.sqrt(var + 1e-5) * g_ref[...] + bb_ref[...]


def _node_update(h, partials, W1n, b1n, W2n, b2n, gamma, beta):
    BN = 2000
    grid = (N // BN,)
    w1h = W1n[:H]
    w1a = W1n[H:]
    return pl.pallas_call(
        _node_body,
        grid=grid,
        in_specs=[
            pl.BlockSpec((BN, H), lambda i: (i, 0)),
            pl.BlockSpec((NC, BN, H), lambda i: (0, i, 0)),
            pl.BlockSpec((H, H), lambda i: (0, 0)),
            pl.BlockSpec((H, H), lambda i: (0, 0)),
            pl.BlockSpec((1, H), lambda i: (0, 0)),
            pl.BlockSpec((H, H), lambda i: (0, 0)),
            pl.BlockSpec((1, H), lambda i: (0, 0)),
            pl.BlockSpec((1, H), lambda i: (0, 0)),
            pl.BlockSpec((1, H), lambda i: (0, 0)),
        ],
        out_specs=pl.BlockSpec((BN, H), lambda i: (i, 0)),
        out_shape=jax.ShapeDtypeStruct((N, H), jnp.float32),
    )(h, partials, w1h, w1a, b1n.reshape(1, H), W2n, b2n.reshape(1, H),
      gamma.reshape(1, H), beta.reshape(1, H))


def kernel(h, edge_index, edge_attr, edge_sh,
           W1e, b1e, W2e, b2e, W1n, b1n, W2n, b2n, gamma, beta):
    edge_in = jnp.concatenate([edge_attr, edge_sh], axis=1)
    src_blk = edge_index[0].reshape(NW, NIB, IBLK, CHUNK)
    dst_blk = edge_index[1].reshape(NW, NIB, IBLK, CHUNK)

    w_edge = _edge_mlp(edge_in, W1e, b1e, W2e, b2e)
    partials = _sc_messages(h, src_blk, dst_blk, w_edge)
    return _node_update(h, partials, W1n, b1n, W2n, b2n, gamma, beta)


# speedup vs baseline: 2.6934x; 1.4302x over previous
import functools

import jax
import jax.numpy as jnp
from jax import lax
from jax.experimental import pallas as pl
from jax.experimental.pallas import tpu as pltpu
from jax.experimental.pallas import tpu_sc as plsc

N = 10000
E = 320000
H = 128
NC = 2
NS = 16
NW = NC * NS
EPW = E // NW
CHUNK = 80
NCHUNK = EPW // CHUNK
IBLK = 25
NIB = NCHUNK // IBLK
RPT = 624
REM = N - RPT * NS

_DN = (((1,), (0,)), ((), ()))


def _edge_mlp_body(ea_ref, es_ref, w1_ref, b1_ref, w2_ref, b2_ref, o_ref):
    t = lax.dot_general(ea_ref[...], w1_ref[:16], _DN,
                        preferred_element_type=jnp.float32)
    t = t + lax.dot_general(es_ref[...], w1_ref[16:20], _DN,
                            preferred_element_type=jnp.float32)
    t = t + b1_ref[...]
    t = t * jax.nn.sigmoid(t)
    u = lax.dot_general(t, w2_ref[...], _DN,
                        preferred_element_type=jnp.float32)
    o_ref[...] = u + b2_ref[...]


def _edge_mlp(edge_attr, edge_sh, W1e, b1e, W2e, b2e):
    BE = 4000
    grid = (E // BE,)
    return pl.pallas_call(
        _edge_mlp_body,
        grid=grid,
        in_specs=[
            pl.BlockSpec((BE, 16), lambda i: (i, 0)),
            pl.BlockSpec((BE, 4), lambda i: (i, 0)),
            pl.BlockSpec(W1e.shape, lambda i: (0, 0)),
            pl.BlockSpec((1, H), lambda i: (0, 0)),
            pl.BlockSpec(W2e.shape, lambda i: (0, 0)),
            pl.BlockSpec((1, H), lambda i: (0, 0)),
        ],
        out_specs=pl.BlockSpec((BE, H), lambda i: (i, 0)),
        out_shape=jax.ShapeDtypeStruct((E, H), jnp.float32),
    )(edge_attr, edge_sh, W1e, b1e.reshape(1, H), W2e, b2e.reshape(1, H))


def _sc_messages(h, src_blk, dst_blk, w_edge):
    mesh = plsc.VectorSubcoreMesh(core_axis_name="core", subcore_axis_name="subcore")

    @functools.partial(
        pl.kernel,
        out_type=jax.ShapeDtypeStruct((NC, N, H), jnp.float32),
        mesh=mesh,
        scratch_types=[
            pltpu.VMEM((IBLK, CHUNK), jnp.int32),
            pltpu.VMEM((IBLK, CHUNK), jnp.int32),
            pltpu.VMEM((CHUNK, H), jnp.float32),
            pltpu.VMEM((CHUNK, H), jnp.float32),
            pltpu.VMEM_SHARED((N, H), jnp.float32),
            pltpu.SemaphoreType.DMA,
            pltpu.SemaphoreType.DMA,
        ],
    )
    def k(h_hbm, src_hbm, dst_hbm, we_hbm, out_hbm,
          src_v, dst_v, rows_v, wv, agg, sem1, sem2):
        cid = lax.axis_index("core")
        sid = lax.axis_index("subcore")
        wid = cid * NS + sid

        @pl.loop(0, CHUNK)
        def _(i):
            for kk in range(H // 16):
                rows_v[i, pl.ds(kk * 16, 16)] = jnp.zeros((16,), jnp.float32)

        zbase = pl.multiple_of(sid * RPT, 8)
        for r in range(7):
            pltpu.sync_copy(rows_v, agg.at[pl.ds(zbase + r * CHUNK, CHUNK)])
        pltpu.sync_copy(rows_v.at[pl.ds(0, RPT - 7 * CHUNK)],
                        agg.at[pl.ds(zbase + 7 * CHUNK, RPT - 7 * CHUNK)])

        @pl.when(sid == NS - 1)
        def _():
            pltpu.sync_copy(rows_v.at[pl.ds(0, REM)],
                            agg.at[pl.ds(NS * RPT, REM)])

        plsc.subcore_barrier()

        for b in range(NIB):
            pltpu.sync_copy(src_hbm.at[wid, b], src_v)
            pltpu.sync_copy(dst_hbm.at[wid, b], dst_v)

            @pl.loop(0, IBLK)
            def _(jj):
                cp1 = pltpu.async_copy(h_hbm.at[src_v.at[jj]], rows_v, sem1)
                cp2 = pltpu.async_copy(
                    we_hbm.at[pl.ds(wid * EPW + b * IBLK * CHUNK + jj * CHUNK,
                                    CHUNK)],
                    wv, sem2)
                cp1.wait()
                cp2.wait()

                @pl.loop(0, CHUNK)
                def _(e):
                    for kk in range(H // 16):
                        s = pl.ds(kk * 16, 16)
                        rows_v[e, s] = rows_v[e, s] * wv[e, s]

                pltpu.sync_copy(rows_v, agg.at[dst_v.at[jj]], add=True)

        plsc.subcore_barrier()
        sl = pl.ds(zbase, RPT)
        pltpu.sync_copy(agg.at[sl], out_hbm.at[cid, sl])

        @pl.when(sid == NS - 1)
        def _():
            slr = pl.ds(NS * RPT, REM)
            pltpu.sync_copy(agg.at[slr], out_hbm.at[cid, slr])

    return k(h, src_blk, dst_blk, w_edge)


def _node_body(h_ref, p_ref, w1_ref, b1_ref, w2_ref, b2_ref,
               g_ref, bb_ref, o_ref):
    hb = h_ref[...]
    agg = p_ref[0] + p_ref[1]
    t = lax.dot_general(hb, w1_ref[:H], _DN,
                        preferred_element_type=jnp.float32)
    t = t + lax.dot_general(agg, w1_ref[H:], _DN,
                            preferred_element_type=jnp.float32)
    t = t + b1_ref[...]
    t = t * jax.nn.sigmoid(t)
    u = lax.dot_general(t, w2_ref[...], _DN,
                        preferred_element_type=jnp.float32)
    x = hb + u + b2_ref[...]
    mean = jnp.mean(x, axis=1, keepdims=True)
    var = jnp.mean((x - mean) ** 2, axis=1, keepdims=True)
    o_ref[...] = (x - mean) / jnp.sqrt(var + 1e-5) * g_ref[...] + bb_ref[...]


def _node_update(h, partials, W1n, b1n, W2n, b2n, gamma, beta):
    BN = 2000
    grid = (N // BN,)
    return pl.pallas_call(
        _node_body,
        grid=grid,
        in_specs=[
            pl.BlockSpec((BN, H), lambda i: (i, 0)),
            pl.BlockSpec((NC, BN, H), lambda i: (0, i, 0)),
            pl.BlockSpec((2 * H, H), lambda i: (0, 0)),
            pl.BlockSpec((1, H), lambda i: (0, 0)),
            pl.BlockSpec((H, H), lambda i: (0, 0)),
            pl.BlockSpec((1, H), lambda i: (0, 0)),
            pl.BlockSpec((1, H), lambda i: (0, 0)),
            pl.BlockSpec((1, H), lambda i: (0, 0)),
        ],
        out_specs=pl.BlockSpec((BN, H), lambda i: (i, 0)),
        out_shape=jax.ShapeDtypeStruct((N, H), jnp.float32),
    )(h, partials, W1n, b1n.reshape(1, H), W2n, b2n.reshape(1, H),
      gamma.reshape(1, H), beta.reshape(1, H))


def kernel(h, edge_index, edge_attr, edge_sh,
           W1e, b1e, W2e, b2e, W1n, b1n, W2n, b2n, gamma, beta):
    src_blk = edge_index[0].reshape(NW, NIB, IBLK, CHUNK)
    dst_blk = edge_index[1].reshape(NW, NIB, IBLK, CHUNK)

    w_edge = _edge_mlp(edge_attr, edge_sh, W1e, b1e, W2e, b2e)
    partials = _sc_messages(h, src_blk, dst_blk, w_edge)
    return _node_update(h, partials, W1n, b1n, W2n, b2n, gamma, beta)
